# Initial kernel scaffold; baseline (speedup 1.0000x reference)
import jax
import jax.numpy as jnp
from jax.experimental import pallas as pl


def kernel(x, edge_index, edge_attr, batch, W_proj, b_proj, W1, b1, W2, b2, root, conv_bias, gru_W_ih, gru_W_hh, gru_b_ih, gru_b_hh, lstm_W_ih, lstm_W_hh, lstm_b_ih, lstm_b_hh, W_fc1, b_fc1, W_fc2, b_fc2):
    raise NotImplementedError("write your pallas kernel here")


# speedup vs baseline: 2.5077x; 2.5077x over previous
import functools

import jax
import jax.numpy as jnp
from jax import lax
from jax.experimental import pallas as pl
from jax.experimental.pallas import tpu as pltpu
from jax.experimental.pallas import tpu_sc as plsc

N = 10000
E = 160000
D_NODE = 128
D_EDGE = 16
H = 32
H_EDGE = 64
B = 64
NUM_LAYERS = 3
S2S_STEPS = 3

NW = 32
PER_W = E // NW
CHUNK = 40
NCHUNK = PER_W // CHUNK
MROWS = 1000
NOUTER = PER_W // MROWS
NINNER = MROWS // CHUNK
STRIPE = N // 16


def _relu(v):
    return jnp.maximum(v, 0.0)


def _linear_body(x_ref, w_ref, b_ref, o_ref, *, relu):
    y = jnp.dot(x_ref[...], w_ref[...], preferred_element_type=jnp.float32)
    y = y + b_ref[...]
    o_ref[...] = _relu(y) if relu else y


def _linear(x, w_t, b, block_rows, relu=True):
    rows, din = x.shape
    dout = w_t.shape[1]
    grid = rows // block_rows
    return pl.pallas_call(
        functools.partial(_linear_body, relu=relu),
        grid=(grid,),
        in_specs=[
            pl.BlockSpec((block_rows, din), lambda i: (i, 0)),
            pl.BlockSpec((din, dout), lambda i: (0, 0)),
            pl.BlockSpec((1, dout), lambda i: (0, 0)),
        ],
        out_specs=pl.BlockSpec((block_rows, dout), lambda i: (i, 0)),
        out_shape=jax.ShapeDtypeStruct((rows, dout), jnp.float32),
    )(x, w_t, b.reshape(1, dout))


def _sc_gather(h, src_resh):
    mesh = plsc.VectorSubcoreMesh(core_axis_name="c", subcore_axis_name="s")

    @functools.partial(
        pl.kernel,
        mesh=mesh,
        compiler_params=pltpu.CompilerParams(use_tc_tiling_on_sc=False),
        out_type=jax.ShapeDtypeStruct((E, H), jnp.float32),
        scratch_types=[
            pltpu.VMEM((NCHUNK, CHUNK), jnp.int32),
            pltpu.VMEM((CHUNK, H), jnp.float32),
            pltpu.SemaphoreType.DMA,
        ],
    )
    def k(h_hbm, src_hbm, out_hbm, idx_v, rows_v, sem):
        cid = lax.axis_index("c")
        sid = lax.axis_index("s")
        w = cid * 16 + sid
        pltpu.sync_copy(src_hbm.at[w], idx_v)

        def body(j, carry):
            pltpu.async_copy(h_hbm.at[idx_v.at[j]], rows_v, sem).wait()
            pltpu.sync_copy(rows_v, out_hbm.at[pl.ds(w * PER_W + j * CHUNK, CHUNK)])
            return carry

        lax.fori_loop(0, NCHUNK, body, 0)

    return k(h, src_resh)


def _sc_scatter(msg, dst_resh, zeros_nh):
    mesh = plsc.VectorSubcoreMesh(core_axis_name="c", subcore_axis_name="s")

    @functools.partial(
        pl.kernel,
        mesh=mesh,
        compiler_params=pltpu.CompilerParams(use_tc_tiling_on_sc=False),
        out_type=jax.ShapeDtypeStruct((2, N, H), jnp.float32),
        scratch_types=[
            pltpu.VMEM((NCHUNK, CHUNK), jnp.int32),
            pltpu.VMEM((MROWS, H), jnp.float32),
            pltpu.VMEM((STRIPE, H), jnp.float32),
            pltpu.VMEM_SHARED((N, H), jnp.float32),
        ],
    )
    def k(msg_hbm, dst_hbm, zero_hbm, out_hbm, dstv, mv, wbv, aggr_sh):
        cid = lax.axis_index("c")
        sid = lax.axis_index("s")
        w = cid * 16 + sid

        @pl.when(sid == 0)
        def _():
            pltpu.sync_copy(zero_hbm, aggr_sh)

        plsc.subcore_barrier()
        pltpu.sync_copy(dst_hbm.at[w], dstv)

        def outer(c, carry):
            pltpu.sync_copy(msg_hbm.at[pl.ds(w * PER_W + c * MROWS, MROWS)], mv)

            def inner(j, carry2):
                pltpu.sync_copy(
                    mv.at[pl.ds(j * CHUNK, CHUNK)],
                    aggr_sh.at[dstv.at[c * NINNER + j]],
                    add=True,
                )
                return carry2

            lax.fori_loop(0, NINNER, inner, 0)
            return carry

        lax.fori_loop(0, NOUTER, outer, 0)
        plsc.subcore_barrier()
        pltpu.sync_copy(aggr_sh.at[pl.ds(sid * STRIPE, STRIPE)], wbv)
        pltpu.sync_copy(wbv, out_hbm.at[cid, pl.ds(sid * STRIPE, STRIPE)])

    return k(msg, dst_resh, zeros_nh)


def _msg_body(eh_ref, gs_ref, w2t_ref, b2_ref, r_ref, s_ref, o_ref):
    ew = jnp.dot(eh_ref[...], w2t_ref[...], preferred_element_type=jnp.float32)
    ew = ew + b2_ref[...]
    gr = jnp.dot(gs_ref[...], r_ref[...], preferred_element_type=jnp.float32)
    o_ref[...] = jnp.dot(gr * ew, s_ref[...], preferred_element_type=jnp.float32)


def _msg(eh, gs, w2t, b2, r_mat, s_mat, block_rows=2000):
    grid = E // block_rows
    hh = H * H
    return pl.pallas_call(
        _msg_body,
        grid=(grid,),
        in_specs=[
            pl.BlockSpec((block_rows, H_EDGE), lambda i: (i, 0)),
            pl.BlockSpec((block_rows, H), lambda i: (i, 0)),
            pl.BlockSpec((H_EDGE, hh), lambda i: (0, 0)),
            pl.BlockSpec((1, hh), lambda i: (0, 0)),
            pl.BlockSpec((H, hh), lambda i: (0, 0)),
            pl.BlockSpec((hh, H), lambda i: (0, 0)),
        ],
        out_specs=pl.BlockSpec((block_rows, H), lambda i: (i, 0)),
        out_shape=jax.ShapeDtypeStruct((E, H), jnp.float32),
    )(eh, gs, w2t, b2.reshape(1, hh), r_mat, s_mat)


def _gru_body(h_ref, pp_ref, root_ref, cb_ref, wih_ref, bih_ref, whh_ref,
              bhh_ref, o_ref):
    h = h_ref[...]
    aggr = pp_ref[0] + pp_ref[1]
    out = jnp.dot(h, root_ref[...], preferred_element_type=jnp.float32)
    out = _relu(out + aggr + cb_ref[...])
    gi = jnp.dot(out, wih_ref[...], preferred_element_type=jnp.float32) + bih_ref[...]
    gh = jnp.dot(h, whh_ref[...], preferred_element_type=jnp.float32) + bhh_ref[...]
    r = jax.nn.sigmoid(gi[:, :H] + gh[:, :H])
    z = jax.nn.sigmoid(gi[:, H:2 * H] + gh[:, H:2 * H])
    n = jnp.tanh(gi[:, 2 * H:] + r * gh[:, 2 * H:])
    o_ref[...] = (1.0 - z) * n + z * h


def _gru(h, parts, root, conv_bias, wih_t, bih, whh_t, bhh):
    return pl.pallas_call(
        _gru_body,
        out_shape=jax.ShapeDtypeStruct((N, H), jnp.float32),
    )(h, parts, root, conv_bias.reshape(1, H), wih_t, bih.reshape(1, 3 * H),
      whh_t, bhh.reshape(1, 3 * H))


def _s2s_body(h_ref, b_ref, wih_ref, bih_ref, whh_ref, bhh_ref, wf1_ref,
              bf1_ref, wf2_ref, bf2_ref, o_ref):
    h = h_ref[...]
    bidx = b_ref[...]
    mt = (bidx == lax.broadcasted_iota(jnp.int32, (N, B), 1)).astype(jnp.float32)
    q_star = jnp.zeros((B, 2 * H), jnp.float32)
    hs = jnp.zeros((B, H), jnp.float32)
    cs = jnp.zeros((B, H), jnp.float32)
    for _ in range(S2S_STEPS):
        gates = (jnp.dot(q_star, wih_ref[...], preferred_element_type=jnp.float32)
                 + bih_ref[...]
                 + jnp.dot(hs, whh_ref[...], preferred_element_type=jnp.float32)
                 + bhh_ref[...])
        ig = jax.nn.sigmoid(gates[:, :H])
        fg = jax.nn.sigmoid(gates[:, H:2 * H])
        gg = jnp.tanh(gates[:, 2 * H:3 * H])
        og = jax.nn.sigmoid(gates[:, 3 * H:])
        cs = fg * cs + ig * gg
        hs = og * jnp.tanh(cs)
        q = hs
        hq = lax.dot_general(h, q, (((1,), (1,)), ((), ())),
                             preferred_element_type=jnp.float32)
        e = jnp.sum(mt * hq, axis=1, keepdims=True)
        emax = jnp.max(jnp.where(mt > 0.0, e, -1e30), axis=0, keepdims=True)
        emax_g = jnp.sum(mt * emax, axis=1, keepdims=True)
        ee = jnp.exp(e - emax_g)
        denom = jnp.sum(mt * ee, axis=0, keepdims=True)
        denom_g = jnp.sum(mt * denom, axis=1, keepdims=True)
        a = ee / (denom_g + 1e-16)
        r_out = lax.dot_general(mt * a, h, (((0,), (0,)), ((), ())),
                                preferred_element_type=jnp.float32)
        q_star = jnp.concatenate([q, r_out], axis=1)
    g = _relu(jnp.dot(q_star, wf1_ref[...], preferred_element_type=jnp.float32)
              + bf1_ref[...])
    o_ref[...] = jnp.dot(g, wf2_ref[...], preferred_element_type=jnp.float32) + bf2_ref[...]


def _s2s(h, batch_col, lstm_wih_t, lstm_bih, lstm_whh_t, lstm_bhh,
         wf1_t, bf1, wf2_t, bf2):
    return pl.pallas_call(
        _s2s_body,
        out_shape=jax.ShapeDtypeStruct((B, 1), jnp.float32),
    )(h, batch_col, lstm_wih_t, lstm_bih.reshape(1, 4 * H), lstm_whh_t,
      lstm_bhh.reshape(1, 4 * H), wf1_t, bf1.reshape(1, H), wf2_t,
      bf2.reshape(1, 1))


def kernel(x, edge_index, edge_attr, batch, W_proj, b_proj, W1, b1, W2, b2,
           root, conv_bias, gru_W_ih, gru_W_hh, gru_b_ih, gru_b_hh,
           lstm_W_ih, lstm_W_hh, lstm_b_ih, lstm_b_hh,
           W_fc1, b_fc1, W_fc2, b_fc2):
    src = edge_index[0].astype(jnp.int32).reshape(NW, NCHUNK, CHUNK)
    dst = edge_index[1].astype(jnp.int32).reshape(NW, NCHUNK, CHUNK)
    batch_col = batch.astype(jnp.int32).reshape(N, 1)
    zeros_nh = jnp.zeros((N, H), jnp.float32)

    r_mat = jnp.kron(jnp.eye(H, dtype=jnp.float32),
                     jnp.ones((1, H), jnp.float32))
    s_mat = jnp.tile(jnp.eye(H, dtype=jnp.float32), (H, 1))

    h = _linear(x, W_proj.T, b_proj, block_rows=2000, relu=True)
    eh = _linear(edge_attr, W1.T, b1, block_rows=10000, relu=True)

    w2t = W2.T
    wih_t = gru_W_ih.T
    whh_t = gru_W_hh.T
    for _ in range(NUM_LAYERS):
        gs = _sc_gather(h, src)
        msg = _msg(eh, gs, w2t, b2, r_mat, s_mat)
        parts = _sc_scatter(msg, dst, zeros_nh)
        h = _gru(h, parts, root, conv_bias, wih_t, gru_b_ih, whh_t, gru_b_hh)

    return _s2s(h, batch_col, lstm_W_ih.T, lstm_b_ih, lstm_W_hh.T, lstm_b_hh,
                W_fc1.T, b_fc1, W_fc2.T, b_fc2)


# speedup vs baseline: 2.7459x; 1.0950x over previous
import functools

import jax
import jax.numpy as jnp
from jax import lax
from jax.experimental import pallas as pl
from jax.experimental.pallas import tpu as pltpu
from jax.experimental.pallas import tpu_sc as plsc

N = 10000
E = 160000
D_NODE = 128
D_EDGE = 16
H = 32
H_EDGE = 64
B = 64
NUM_LAYERS = 3
S2S_STEPS = 3

NW = 32
PER_W = E // NW
CHUNK = 50
NCHUNK = PER_W // CHUNK
MROWS = 1000
NOUTER = PER_W // MROWS
NINNER = MROWS // CHUNK
STRIPE = N // 16


def _relu(v):
    return jnp.maximum(v, 0.0)


def _linear_body(x_ref, w_ref, b_ref, o_ref, *, relu, out_dtype):
    y = jnp.dot(x_ref[...], w_ref[...], preferred_element_type=jnp.float32)
    y = y + b_ref[...]
    y = _relu(y) if relu else y
    o_ref[...] = y.astype(out_dtype)


def _linear(x, w_t, b, block_rows, relu=True, out_dtype=jnp.float32):
    rows, din = x.shape
    dout = w_t.shape[1]
    grid = rows // block_rows
    return pl.pallas_call(
        functools.partial(_linear_body, relu=relu, out_dtype=out_dtype),
        grid=(grid,),
        in_specs=[
            pl.BlockSpec((block_rows, din), lambda i: (i, 0)),
            pl.BlockSpec((din, dout), lambda i: (0, 0)),
            pl.BlockSpec((1, dout), lambda i: (0, 0)),
        ],
        out_specs=pl.BlockSpec((block_rows, dout), lambda i: (i, 0)),
        out_shape=jax.ShapeDtypeStruct((rows, dout), out_dtype),
    )(x, w_t, b.reshape(1, dout))


def _sc_gather(h, src_resh):
    mesh = plsc.VectorSubcoreMesh(core_axis_name="c", subcore_axis_name="s")

    @functools.partial(
        pl.kernel,
        mesh=mesh,
        compiler_params=pltpu.CompilerParams(use_tc_tiling_on_sc=False),
        out_type=jax.ShapeDtypeStruct((E, H), jnp.float32),
        scratch_types=[
            pltpu.VMEM((NCHUNK, CHUNK), jnp.int32),
            pltpu.VMEM((MROWS, H), jnp.float32),
            pltpu.SemaphoreType.DMA,
        ],
    )
    def k(h_hbm, src_hbm, out_hbm, idx_v, rows_v, sem):
        cid = lax.axis_index("c")
        sid = lax.axis_index("s")
        w = cid * 16 + sid
        pltpu.sync_copy(src_hbm.at[w], idx_v)

        def body(c, carry):
            descs = [
                pltpu.async_copy(
                    h_hbm.at[idx_v.at[c * NINNER + jj]],
                    rows_v.at[pl.ds(jj * CHUNK, CHUNK)],
                    sem,
                )
                for jj in range(NINNER)
            ]
            for d in descs:
                d.wait()
            pltpu.sync_copy(rows_v, out_hbm.at[pl.ds(w * PER_W + c * MROWS, MROWS)])
            return carry

        lax.fori_loop(0, NOUTER, body, 0)

    return k(h, src_resh)


def _sc_scatter(msg, dst_resh, zeros_nh):
    mesh = plsc.VectorSubcoreMesh(core_axis_name="c", subcore_axis_name="s")

    @functools.partial(
        pl.kernel,
        mesh=mesh,
        compiler_params=pltpu.CompilerParams(use_tc_tiling_on_sc=False),
        out_type=jax.ShapeDtypeStruct((2, N, H), jnp.float32),
        scratch_types=[
            pltpu.VMEM((NCHUNK, CHUNK), jnp.int32),
            pltpu.VMEM((MROWS, H), jnp.float32),
            pltpu.VMEM((STRIPE, H), jnp.float32),
            pltpu.VMEM_SHARED((N, H), jnp.float32),
        ],
    )
    def k(msg_hbm, dst_hbm, zero_hbm, out_hbm, dstv, mv, wbv, aggr_sh):
        cid = lax.axis_index("c")
        sid = lax.axis_index("s")
        w = cid * 16 + sid

        @pl.when(sid == 0)
        def _():
            pltpu.sync_copy(zero_hbm, aggr_sh)

        plsc.subcore_barrier()
        pltpu.sync_copy(dst_hbm.at[w], dstv)

        def outer(c, carry):
            pltpu.sync_copy(msg_hbm.at[pl.ds(w * PER_W + c * MROWS, MROWS)], mv)

            def inner(j, carry2):
                pltpu.sync_copy(
                    mv.at[pl.ds(j * CHUNK, CHUNK)],
                    aggr_sh.at[dstv.at[c * NINNER + j]],
                    add=True,
                )
                return carry2

            lax.fori_loop(0, NINNER, inner, 0)
            return carry

        lax.fori_loop(0, NOUTER, outer, 0)
        plsc.subcore_barrier()
        pltpu.sync_copy(aggr_sh.at[pl.ds(sid * STRIPE, STRIPE)], wbv)
        pltpu.sync_copy(wbv, out_hbm.at[cid, pl.ds(sid * STRIPE, STRIPE)])

    return k(msg, dst_resh, zeros_nh)


def _msg_body(eh_ref, gs_ref, w2t_ref, b2_ref, r_ref, s_ref, o_ref):
    ew = jnp.dot(eh_ref[...], w2t_ref[...],
                 preferred_element_type=jnp.float32) + b2_ref[...]
    gr = jnp.dot(gs_ref[...], r_ref[...], preferred_element_type=jnp.float32)
    o_ref[...] = jnp.dot(gr * ew, s_ref[...],
                         preferred_element_type=jnp.float32)


def _msg(eh, gs, w2t, b2, r_mat, s_mat, block_rows=2000):
    grid = E // block_rows
    hh = H * H
    return pl.pallas_call(
        _msg_body,
        grid=(grid,),
        in_specs=[
            pl.BlockSpec((block_rows, H_EDGE), lambda i: (i, 0)),
            pl.BlockSpec((block_rows, H), lambda i: (i, 0)),
            pl.BlockSpec((H_EDGE, hh), lambda i: (0, 0)),
            pl.BlockSpec((1, hh), lambda i: (0, 0)),
            pl.BlockSpec((H, hh), lambda i: (0, 0)),
            pl.BlockSpec((hh, H), lambda i: (0, 0)),
        ],
        out_specs=pl.BlockSpec((block_rows, H), lambda i: (i, 0)),
        out_shape=jax.ShapeDtypeStruct((E, H), jnp.float32),
    )(eh, gs, w2t, b2.reshape(1, hh), r_mat, s_mat)


def _gru_body(h_ref, pp_ref, root_ref, cb_ref, wih_ref, bih_ref, whh_ref,
              bhh_ref, o_ref):
    h = h_ref[...]
    aggr = pp_ref[0] + pp_ref[1]
    out = jnp.dot(h, root_ref[...], preferred_element_type=jnp.float32)
    out = _relu(out + aggr + cb_ref[...])
    gi = jnp.dot(out, wih_ref[...], preferred_element_type=jnp.float32) + bih_ref[...]
    gh = jnp.dot(h, whh_ref[...], preferred_element_type=jnp.float32) + bhh_ref[...]
    r = jax.nn.sigmoid(gi[:, :H] + gh[:, :H])
    z = jax.nn.sigmoid(gi[:, H:2 * H] + gh[:, H:2 * H])
    n = jnp.tanh(gi[:, 2 * H:] + r * gh[:, 2 * H:])
    o_ref[...] = (1.0 - z) * n + z * h


def _gru(h, parts, root, conv_bias, wih_t, bih, whh_t, bhh):
    return pl.pallas_call(
        _gru_body,
        out_shape=jax.ShapeDtypeStruct((N, H), jnp.float32),
    )(h, parts, root, conv_bias.reshape(1, H), wih_t, bih.reshape(1, 3 * H),
      whh_t, bhh.reshape(1, 3 * H))


def _s2s_body(h_ref, b_ref, wih_ref, bih_ref, whh_ref, bhh_ref, wf1_ref,
              bf1_ref, wf2_ref, bf2_ref, o_ref):
    h = h_ref[...]
    bidx = b_ref[...]
    mt = (bidx == lax.broadcasted_iota(jnp.int32, (N, B), 1)).astype(jnp.float32)
    q_star = jnp.zeros((B, 2 * H), jnp.float32)
    hs = jnp.zeros((B, H), jnp.float32)
    cs = jnp.zeros((B, H), jnp.float32)
    for _ in range(S2S_STEPS):
        gates = (jnp.dot(q_star, wih_ref[...], preferred_element_type=jnp.float32)
                 + bih_ref[...]
                 + jnp.dot(hs, whh_ref[...], preferred_element_type=jnp.float32)
                 + bhh_ref[...])
        ig = jax.nn.sigmoid(gates[:, :H])
        fg = jax.nn.sigmoid(gates[:, H:2 * H])
        gg = jnp.tanh(gates[:, 2 * H:3 * H])
        og = jax.nn.sigmoid(gates[:, 3 * H:])
        cs = fg * cs + ig * gg
        hs = og * jnp.tanh(cs)
        q = hs
        hq = lax.dot_general(h, q, (((1,), (1,)), ((), ())),
                             preferred_element_type=jnp.float32)
        e = jnp.sum(mt * hq, axis=1, keepdims=True)
        emax = jnp.max(jnp.where(mt > 0.0, e, -1e30), axis=0, keepdims=True)
        emax_g = jnp.sum(mt * emax, axis=1, keepdims=True)
        ee = jnp.exp(e - emax_g)
        denom = jnp.sum(mt * ee, axis=0, keepdims=True)
        denom_g = jnp.sum(mt * denom, axis=1, keepdims=True)
        a = ee / (denom_g + 1e-16)
        r_out = lax.dot_general(mt * a, h, (((0,), (0,)), ((), ())),
                                preferred_element_type=jnp.float32)
        q_star = jnp.concatenate([q, r_out], axis=1)
    g = _relu(jnp.dot(q_star, wf1_ref[...], preferred_element_type=jnp.float32)
              + bf1_ref[...])
    o_ref[...] = jnp.dot(g, wf2_ref[...], preferred_element_type=jnp.float32) + bf2_ref[...]


def _s2s(h, batch_col, lstm_wih_t, lstm_bih, lstm_whh_t, lstm_bhh,
         wf1_t, bf1, wf2_t, bf2):
    return pl.pallas_call(
        _s2s_body,
        out_shape=jax.ShapeDtypeStruct((B, 1), jnp.float32),
    )(h, batch_col, lstm_wih_t, lstm_bih.reshape(1, 4 * H), lstm_whh_t,
      lstm_bhh.reshape(1, 4 * H), wf1_t, bf1.reshape(1, H), wf2_t,
      bf2.reshape(1, 1))


def kernel(x, edge_index, edge_attr, batch, W_proj, b_proj, W1, b1, W2, b2,
           root, conv_bias, gru_W_ih, gru_W_hh, gru_b_ih, gru_b_hh,
           lstm_W_ih, lstm_W_hh, lstm_b_ih, lstm_b_hh,
           W_fc1, b_fc1, W_fc2, b_fc2):
    src = edge_index[0].astype(jnp.int32).reshape(NW, NCHUNK, CHUNK)
    dst = edge_index[1].astype(jnp.int32).reshape(NW, NCHUNK, CHUNK)
    batch_col = batch.astype(jnp.int32).reshape(N, 1)
    zeros_nh = jnp.zeros((N, H), jnp.float32)

    r_mat = jnp.kron(jnp.eye(H, dtype=jnp.float32),
                     jnp.ones((1, H), jnp.float32))
    s_mat = jnp.tile(jnp.eye(H, dtype=jnp.float32), (H, 1))

    h = _linear(x, W_proj.T, b_proj, block_rows=2000, relu=True)
    eh = _linear(edge_attr, W1.T, b1, block_rows=10000, relu=True)

    w2t = W2.T
    wih_t = gru_W_ih.T
    whh_t = gru_W_hh.T
    for _ in range(NUM_LAYERS):
        gs = _sc_gather(h, src)
        msg = _msg(eh, gs, w2t, b2, r_mat, s_mat)
        parts = _sc_scatter(msg, dst, zeros_nh)
        h = _gru(h, parts, root, conv_bias, wih_t, gru_b_ih, whh_t, gru_b_hh)

    return _s2s(h, batch_col, lstm_W_ih.T, lstm_b_ih, lstm_W_hh.T, lstm_b_hh,
                W_fc1.T, b_fc1, W_fc2.T, b_fc2)


# speedup vs baseline: 3.2574x; 1.1863x over previous
import functools

import jax
import jax.numpy as jnp
from jax import lax
from jax.experimental import pallas as pl
from jax.experimental.pallas import tpu as pltpu
from jax.experimental.pallas import tpu_sc as plsc

N = 10000
E = 160000
D_NODE = 128
D_EDGE = 16
H = 32
H_EDGE = 64
B = 64
NUM_LAYERS = 3
S2S_STEPS = 3

NW = 32
PER_W = E // NW
CHUNK = 50
NCHUNK = PER_W // CHUNK
MROWS = 1000
NOUTER = PER_W // MROWS
NINNER = MROWS // CHUNK
STRIPE = N // 16


def _relu(v):
    return jnp.maximum(v, 0.0)


def _linear_body(x_ref, w_ref, b_ref, o_ref, *, relu, out_dtype):
    y = jnp.dot(x_ref[...], w_ref[...], preferred_element_type=jnp.float32)
    y = y + b_ref[...]
    y = _relu(y) if relu else y
    o_ref[...] = y.astype(out_dtype)


def _linear(x, w_t, b, block_rows, relu=True, out_dtype=jnp.float32):
    rows, din = x.shape
    dout = w_t.shape[1]
    grid = rows // block_rows
    return pl.pallas_call(
        functools.partial(_linear_body, relu=relu, out_dtype=out_dtype),
        grid=(grid,),
        in_specs=[
            pl.BlockSpec((block_rows, din), lambda i: (i, 0)),
            pl.BlockSpec((din, dout), lambda i: (0, 0)),
            pl.BlockSpec((1, dout), lambda i: (0, 0)),
        ],
        out_specs=pl.BlockSpec((block_rows, dout), lambda i: (i, 0)),
        out_shape=jax.ShapeDtypeStruct((rows, dout), out_dtype),
    )(x, w_t, b.reshape(1, dout))


def _sc_gather(h, src_resh):
    mesh = plsc.VectorSubcoreMesh(core_axis_name="c", subcore_axis_name="s")

    @functools.partial(
        pl.kernel,
        mesh=mesh,
        compiler_params=pltpu.CompilerParams(use_tc_tiling_on_sc=False),
        out_type=jax.ShapeDtypeStruct((E, H), jnp.float32),
        scratch_types=[
            pltpu.VMEM((NCHUNK, CHUNK), jnp.int32),
            pltpu.VMEM((MROWS, H), jnp.float32),
            pltpu.SemaphoreType.DMA,
        ],
    )
    def k(h_hbm, src_hbm, out_hbm, idx_v, rows_v, sem):
        cid = lax.axis_index("c")
        sid = lax.axis_index("s")
        w = cid * 16 + sid
        pltpu.sync_copy(src_hbm.at[w], idx_v)

        def body(c, carry):
            descs = [
                pltpu.async_copy(
                    h_hbm.at[idx_v.at[c * NINNER + jj]],
                    rows_v.at[pl.ds(jj * CHUNK, CHUNK)],
                    sem,
                )
                for jj in range(NINNER)
            ]
            for d in descs:
                d.wait()
            pltpu.sync_copy(rows_v, out_hbm.at[pl.ds(w * PER_W + c * MROWS, MROWS)])
            return carry

        lax.fori_loop(0, NOUTER, body, 0)

    return k(h, src_resh)


def _sc_scatter(msg, dst_resh, zeros_nh):
    mesh = plsc.VectorSubcoreMesh(core_axis_name="c", subcore_axis_name="s")

    @functools.partial(
        pl.kernel,
        mesh=mesh,
        compiler_params=pltpu.CompilerParams(use_tc_tiling_on_sc=False),
        out_type=jax.ShapeDtypeStruct((2, N, H), jnp.float32),
        scratch_types=[
            pltpu.VMEM((NCHUNK, CHUNK), jnp.int32),
            pltpu.VMEM((MROWS, H), jnp.float32),
            pltpu.VMEM((STRIPE, H), jnp.float32),
            pltpu.VMEM_SHARED((N, H), jnp.float32),
        ],
    )
    def k(msg_hbm, dst_hbm, zero_hbm, out_hbm, dstv, mv, wbv, aggr_sh):
        cid = lax.axis_index("c")
        sid = lax.axis_index("s")
        w = cid * 16 + sid

        @pl.when(sid == 0)
        def _():
            pltpu.sync_copy(zero_hbm, aggr_sh)

        plsc.subcore_barrier()
        pltpu.sync_copy(dst_hbm.at[w], dstv)

        def outer(c, carry):
            pltpu.sync_copy(msg_hbm.at[pl.ds(w * PER_W + c * MROWS, MROWS)], mv)

            def inner(j, carry2):
                pltpu.sync_copy(
                    mv.at[pl.ds(j * CHUNK, CHUNK)],
                    aggr_sh.at[dstv.at[c * NINNER + j]],
                    add=True,
                )
                return carry2

            lax.fori_loop(0, NINNER, inner, 0)
            return carry

        lax.fori_loop(0, NOUTER, outer, 0)
        plsc.subcore_barrier()
        pltpu.sync_copy(aggr_sh.at[pl.ds(sid * STRIPE, STRIPE)], wbv)
        pltpu.sync_copy(wbv, out_hbm.at[cid, pl.ds(sid * STRIPE, STRIPE)])

    return k(msg, dst_resh, zeros_nh)


def _msg_body(eh_ref, gs_ref, w2t_ref, b2_ref, r_ref, s_ref, o_ref):
    ew = jnp.dot(eh_ref[...], w2t_ref[...],
                 preferred_element_type=jnp.float32) + b2_ref[...]
    gr = jnp.dot(gs_ref[...], r_ref[...], preferred_element_type=jnp.float32)
    o_ref[...] = jnp.dot(gr * ew, s_ref[...],
                         preferred_element_type=jnp.float32)


EP = E // 4
PBLK = 1000


def _msg(eh_p, gs_p, w2t_p, b2_p, r_p, s_p):
    grid = EP // PBLK
    return pl.pallas_call(
        _msg_body,
        grid=(grid,),
        in_specs=[
            pl.BlockSpec((PBLK, 4 * H_EDGE), lambda i: (i, 0)),
            pl.BlockSpec((PBLK, 4 * H), lambda i: (i, 0)),
            pl.BlockSpec((4 * H_EDGE, 4 * H * H), lambda i: (0, 0)),
            pl.BlockSpec((1, 4 * H * H), lambda i: (0, 0)),
            pl.BlockSpec((4 * H, 4 * H * H), lambda i: (0, 0)),
            pl.BlockSpec((4 * H * H, 4 * H), lambda i: (0, 0)),
        ],
        out_specs=pl.BlockSpec((PBLK, 4 * H), lambda i: (i, 0)),
        out_shape=jax.ShapeDtypeStruct((EP, 4 * H), jnp.float32),
    )(eh_p, gs_p, w2t_p, b2_p, r_p, s_p)


def _gru_body(h_ref, pp_ref, root_ref, cb_ref, wih_ref, bih_ref, whh_ref,
              bhh_ref, o_ref):
    h = h_ref[...]
    aggr = pp_ref[0] + pp_ref[1]
    out = jnp.dot(h, root_ref[...], preferred_element_type=jnp.float32)
    out = _relu(out + aggr + cb_ref[...])
    gi = jnp.dot(out, wih_ref[...], preferred_element_type=jnp.float32) + bih_ref[...]
    gh = jnp.dot(h, whh_ref[...], preferred_element_type=jnp.float32) + bhh_ref[...]
    r = jax.nn.sigmoid(gi[:, :H] + gh[:, :H])
    z = jax.nn.sigmoid(gi[:, H:2 * H] + gh[:, H:2 * H])
    n = jnp.tanh(gi[:, 2 * H:] + r * gh[:, 2 * H:])
    o_ref[...] = (1.0 - z) * n + z * h


def _gru(h, parts, root, conv_bias, wih_t, bih, whh_t, bhh):
    return pl.pallas_call(
        _gru_body,
        out_shape=jax.ShapeDtypeStruct((N, H), jnp.float32),
    )(h, parts, root, conv_bias.reshape(1, H), wih_t, bih.reshape(1, 3 * H),
      whh_t, bhh.reshape(1, 3 * H))


def _s2s_body(h_ref, b_ref, wih_ref, bih_ref, whh_ref, bhh_ref, wf1_ref,
              bf1_ref, wf2_ref, bf2_ref, o_ref):
    h = h_ref[...]
    bidx = b_ref[...]
    mt = (bidx == lax.broadcasted_iota(jnp.int32, (N, B), 1)).astype(jnp.float32)
    q_star = jnp.zeros((B, 2 * H), jnp.float32)
    hs = jnp.zeros((B, H), jnp.float32)
    cs = jnp.zeros((B, H), jnp.float32)
    for _ in range(S2S_STEPS):
        gates = (jnp.dot(q_star, wih_ref[...], preferred_element_type=jnp.float32)
                 + bih_ref[...]
                 + jnp.dot(hs, whh_ref[...], preferred_element_type=jnp.float32)
                 + bhh_ref[...])
        ig = jax.nn.sigmoid(gates[:, :H])
        fg = jax.nn.sigmoid(gates[:, H:2 * H])
        gg = jnp.tanh(gates[:, 2 * H:3 * H])
        og = jax.nn.sigmoid(gates[:, 3 * H:])
        cs = fg * cs + ig * gg
        hs = og * jnp.tanh(cs)
        q = hs
        hq = lax.dot_general(h, q, (((1,), (1,)), ((), ())),
                             preferred_element_type=jnp.float32)
        e = jnp.sum(mt * hq, axis=1, keepdims=True)
        emax = jnp.max(jnp.where(mt > 0.0, e, -1e30), axis=0, keepdims=True)
        emax_g = jnp.sum(mt * emax, axis=1, keepdims=True)
        ee = jnp.exp(e - emax_g)
        denom = jnp.sum(mt * ee, axis=0, keepdims=True)
        denom_g = jnp.sum(mt * denom, axis=1, keepdims=True)
        a = ee / (denom_g + 1e-16)
        r_out = lax.dot_general(mt * a, h, (((0,), (0,)), ((), ())),
                                preferred_element_type=jnp.float32)
        q_star = jnp.concatenate([q, r_out], axis=1)
    g = _relu(jnp.dot(q_star, wf1_ref[...], preferred_element_type=jnp.float32)
              + bf1_ref[...])
    o_ref[...] = jnp.dot(g, wf2_ref[...], preferred_element_type=jnp.float32) + bf2_ref[...]


def _s2s(h, batch_col, lstm_wih_t, lstm_bih, lstm_whh_t, lstm_bhh,
         wf1_t, bf1, wf2_t, bf2):
    return pl.pallas_call(
        _s2s_body,
        out_shape=jax.ShapeDtypeStruct((B, 1), jnp.float32),
    )(h, batch_col, lstm_wih_t, lstm_bih.reshape(1, 4 * H), lstm_whh_t,
      lstm_bhh.reshape(1, 4 * H), wf1_t, bf1.reshape(1, H), wf2_t,
      bf2.reshape(1, 1))


def kernel(x, edge_index, edge_attr, batch, W_proj, b_proj, W1, b1, W2, b2,
           root, conv_bias, gru_W_ih, gru_W_hh, gru_b_ih, gru_b_hh,
           lstm_W_ih, lstm_W_hh, lstm_b_ih, lstm_b_hh,
           W_fc1, b_fc1, W_fc2, b_fc2):
    src = edge_index[0].astype(jnp.int32).reshape(NW, NCHUNK, CHUNK)
    dst = edge_index[1].astype(jnp.int32).reshape(NW, NCHUNK, CHUNK)
    batch_col = batch.astype(jnp.int32).reshape(N, 1)
    zeros_nh = jnp.zeros((N, H), jnp.float32)

    r_mat = jnp.kron(jnp.eye(H, dtype=jnp.float32),
                     jnp.ones((1, H), jnp.float32))
    s_mat = jnp.tile(jnp.eye(H, dtype=jnp.float32), (H, 1))
    eye4 = jnp.eye(4, dtype=jnp.float32)
    r_p = jnp.kron(eye4, r_mat)
    s_p = jnp.kron(eye4, s_mat)
    w2t_p = jnp.kron(eye4, W2.T)
    b2_p = jnp.tile(b2, 4).reshape(1, 4 * H * H)
    w1t_p = jnp.kron(eye4, W1.T)
    b1_p = jnp.tile(b1, 4)

    h = _linear(x, W_proj.T, b_proj, block_rows=2000, relu=True)
    eh_p = _linear(edge_attr.reshape(EP, 4 * D_EDGE), w1t_p, b1_p,
                   block_rows=2000, relu=True)

    wih_t = gru_W_ih.T
    whh_t = gru_W_hh.T
    for _ in range(NUM_LAYERS):
        gs_p = _sc_gather(h, src).reshape(EP, 4 * H)
        msg_p = _msg(eh_p, gs_p, w2t_p, b2_p, r_p, s_p)
        parts = _sc_scatter(msg_p.reshape(E, H), dst, zeros_nh)
        h = _gru(h, parts, root, conv_bias, wih_t, gru_b_ih, whh_t, gru_b_hh)

    return _s2s(h, batch_col, lstm_W_ih.T, lstm_b_ih, lstm_W_hh.T, lstm_b_hh,
                W_fc1.T, b_fc1, W_fc2.T, b_fc2)
